# baseline (speedup 1.0000x reference)
import functools

import jax
import jax.numpy as jnp
import ml_dtypes
import numpy as np
from jax import lax
from jax.experimental import pallas as pl
from jax.experimental.pallas import tpu as pltpu
from jax.experimental.pallas import tpu_sc as plsc

VOCAB = 100000
D_MODEL = 1024
MAX_SEQ = 2048
BATCH = 4

NUM_CORES = 2
NUM_SUBCORES = 16
NW = NUM_CORES * NUM_SUBCORES
PPW = MAX_SEQ // NW
CHUNK = 16
NPC = PPW // CHUNK
NCHUNK = NPC * BATCH
NBUF = 4
LANES = 16


def _positional_encoding_bf16():
    pos = np.arange(MAX_SEQ, dtype=np.float32)[:, None]
    i = np.arange(0, D_MODEL, 2, dtype=np.float32)
    denom = np.power(np.float32(10000.0), i / np.float32(D_MODEL))
    pe = np.zeros((MAX_SEQ, D_MODEL), dtype=np.float32)
    pe[:, 0::2] = np.sin(pos / denom)
    pe[:, 1::2] = np.cos(pos / denom)
    g = pe.reshape(MAX_SEQ, D_MODEL // 32, 2, LANES)
    inter = np.transpose(g, (0, 1, 3, 2)).reshape(-1)
    pairs = inter.astype(ml_dtypes.bfloat16).view(np.uint32)
    return pairs.view(np.int32)


_PE_PACKED = _positional_encoding_bf16()


def _sc_body(table_hbm, idx_hbm, pe_hbm, out_hbm, idx_v, rows_v, pe_v0, pe_v1,
             gsem0, gsem1, gsem2, gsem3, psem0, psem1,
             wsem0, wsem1, wsem2, wsem3):
    cid = lax.axis_index("c")
    sid = lax.axis_index("s")
    wid = sid * NUM_CORES + cid
    pbase = wid * PPW

    pe_bufs = (pe_v0, pe_v1)
    gsem = (gsem0, gsem1, gsem2, gsem3)
    psem = (psem0, psem1)
    wsem = (wsem0, wsem1, wsem2, wsem3)

    pltpu.sync_copy(idx_hbm.at[wid], idx_v)

    pe_cp = [None] * 2
    gather = [None] * NBUF
    wb = [None] * NBUF

    def start_pe(c):
        q = c % 2
        pe_cp[q] = pltpu.async_copy(
            pe_hbm.at[pl.ds((pbase + c * CHUNK) * (D_MODEL // 2),
                            CHUNK * D_MODEL // 2)],
            pe_bufs[q], psem[q])

    def start_gather(j):
        p = j % NBUF
        if wb[p] is not None:
            wb[p].wait()
            wb[p] = None
        gather[p] = pltpu.async_copy(
            table_hbm.at[idx_v.at[j]], rows_v.at[p], gsem[p])

    start_pe(0)
    start_gather(0)
    start_gather(1)
    for j in range(NCHUNK):
        p = j % NBUF
        c, b = divmod(j, BATCH)
        if j + 2 < NCHUNK:
            start_gather(j + 2)
        if b == 0 and c + 1 < NPC:
            start_pe(c + 1)
        gather[p].wait()
        if b == 0:
            pe_cp[c % 2].wait()
        q = c % 2

        def add_row(r, carry):
            for k in range(D_MODEL // (2 * LANES)):
                pair = pe_bufs[q][pl.ds(r * (D_MODEL // 2) + k * LANES,
                                        LANES)]
                lo = lax.bitcast_convert_type(pair << 16, jnp.float32)
                hi = lax.bitcast_convert_type(pair & jnp.int32(-65536), jnp.float32)
                plsc.addupdate(rows_v.at[p, r, pl.ds(k * 2 * LANES, LANES)],
                               lo)
                plsc.addupdate(
                    rows_v.at[p, r, pl.ds(k * 2 * LANES + LANES, LANES)], hi)
            return carry

        lax.fori_loop(0, CHUNK, add_row, 0)
        wb[p] = pltpu.async_copy(
            rows_v.at[p],
            out_hbm.at[b].at[pl.ds(pbase + c * CHUNK, CHUNK)], wsem[p])
    for w in wb:
        if w is not None:
            w.wait()


@jax.jit
def _embed(x, table):
    idx = (x.astype(jnp.int32)
           .reshape(BATCH, NW, NPC, CHUNK)
           .transpose(1, 2, 0, 3)
           .reshape(NW, NCHUNK, CHUNK))
    mesh = plsc.VectorSubcoreMesh(core_axis_name="c", subcore_axis_name="s")
    gather = functools.partial(
        pl.kernel,
        mesh=mesh,
        out_type=jax.ShapeDtypeStruct((BATCH, MAX_SEQ, D_MODEL), jnp.float32),
        scratch_types=[
            pltpu.VMEM((NCHUNK, CHUNK), jnp.int32),
            pltpu.VMEM((NBUF, CHUNK, D_MODEL), jnp.float32),
            pltpu.VMEM((CHUNK * D_MODEL // 2,), jnp.int32),
            pltpu.VMEM((CHUNK * D_MODEL // 2,), jnp.int32),
        ] + [pltpu.SemaphoreType.DMA] * 10,
    )(_sc_body)
    return gather(table, idx, jnp.asarray(_PE_PACKED))


def kernel(x, table):
    return _embed(x, table)
